# Initial kernel scaffold; baseline (speedup 1.0000x reference)
import functools

import jax
import jax.numpy as jnp
from jax import lax
from jax.experimental import pallas as pl
from jax.experimental.pallas import tpu as pltpu
from jax.experimental.pallas import tpu_sc as plsc

H = 64
IDX_CHUNK = 128


def _sc_gather(table, idx):
    n = idx.shape[0]
    info = plsc.get_sparse_core_info()
    nw = info.num_cores * info.num_subcores
    assert n % nw == 0
    b_per_w = n // nw
    assert b_per_w % 8 == 0
    sizes = []
    left = b_per_w
    while left > 0:
        s = min(IDX_CHUNK, left)
        sizes.append(s)
        left -= s

    mesh = plsc.VectorSubcoreMesh(core_axis_name="c", subcore_axis_name="s")

    @functools.partial(
        pl.kernel,
        out_type=jax.ShapeDtypeStruct((n, H), jnp.float32),
        mesh=mesh,
        scratch_types=[
            pltpu.VMEM((b_per_w,), jnp.int32),
            pltpu.VMEM((b_per_w, H), jnp.float32),
            pltpu.SemaphoreType.DMA,
        ],
    )
    def k(table_hbm, idx_hbm, out_hbm, idx_v, rows_v, sem):
        wid = lax.axis_index("s") * info.num_cores + lax.axis_index("c")
        base = wid * b_per_w
        pltpu.sync_copy(idx_hbm.at[pl.ds(base, b_per_w)], idx_v)
        copies = []
        off = 0
        for s in sizes:
            copies.append(
                pltpu.async_copy(
                    table_hbm.at[idx_v.at[pl.ds(off, s)]],
                    rows_v.at[pl.ds(off, s)],
                    sem,
                )
            )
            off += s
        for c in copies:
            c.wait()
        pltpu.sync_copy(rows_v, out_hbm.at[pl.ds(base, b_per_w)])

    return k(table, idx)


def _lstm_tc(emb_tm, ctx_tm, h0f, c0f, h0b, c0b, Wf, bf, Wb, bb):
    T, B, _ = emb_tm.shape

    def body(emb_f, ctx_f, emb_b, ctx_b, h0f_r, c0f_r, h0b_r, c0b_r,
             wf_r, bf_r, wb_r, bb_r,
             out_f, out_b, hf_o, cf_o, hb_o, cb_o,
             hf_s, cf_s, hb_s, cb_s):
        t = pl.program_id(0)

        @pl.when(t == 0)
        def _():
            hf_s[:] = h0f_r[:]
            cf_s[:] = c0f_r[:]
            hb_s[:] = h0b_r[:]
            cb_s[:] = c0b_r[:]

        def step(emb, ctx, h, c, w, b):
            x = jnp.concatenate([emb, ctx, h], axis=-1)
            g = jnp.dot(x, w, preferred_element_type=jnp.float32) + b
            i = jax.nn.sigmoid(g[:, 0 * H:1 * H])
            f = jax.nn.sigmoid(g[:, 1 * H:2 * H])
            gg = jnp.tanh(g[:, 2 * H:3 * H])
            o = jax.nn.sigmoid(g[:, 3 * H:4 * H])
            c2 = f * c + i * gg
            h2 = o * jnp.tanh(c2)
            return h2, c2

        hf, cf = step(emb_f[0], ctx_f[0], hf_s[:], cf_s[:], wf_r[:], bf_r[:])
        hf_s[:] = hf
        cf_s[:] = cf
        out_f[0] = hf

        hb, cb = step(emb_b[0], ctx_b[0], hb_s[:], cb_s[:], wb_r[:], bb_r[:])
        hb_s[:] = hb
        cb_s[:] = cb
        out_b[0] = hb

        @pl.when(t == T - 1)
        def _():
            hf_o[:] = hf
            cf_o[:] = cf
            hb_o[:] = hb
            cb_o[:] = cb

    full = lambda shape: pl.BlockSpec(shape, lambda t: (0,) * len(shape))
    tspec = lambda w: pl.BlockSpec((1, B, w), lambda t: (t, 0, 0))
    rspec = lambda w: pl.BlockSpec((1, B, w), lambda t: (T - 1 - t, 0, 0))

    grid_spec = pl.GridSpec(
        grid=(T,),
        in_specs=[
            tspec(H), tspec(2 * H), rspec(H), rspec(2 * H),
            full((B, H)), full((B, H)), full((B, H)), full((B, H)),
            full((4 * H, 4 * H)), full((1, 4 * H)),
            full((4 * H, 4 * H)), full((1, 4 * H)),
        ],
        out_specs=[
            tspec(H), rspec(H),
            full((B, H)), full((B, H)), full((B, H)), full((B, H)),
        ],
    )
    out_shape = [
        jax.ShapeDtypeStruct((T, B, H), jnp.float32),
        jax.ShapeDtypeStruct((T, B, H), jnp.float32),
        jax.ShapeDtypeStruct((B, H), jnp.float32),
        jax.ShapeDtypeStruct((B, H), jnp.float32),
        jax.ShapeDtypeStruct((B, H), jnp.float32),
        jax.ShapeDtypeStruct((B, H), jnp.float32),
    ]
    scratch = [pltpu.VMEM((B, H), jnp.float32)] * 4
    return pl.pallas_call(
        body,
        grid_spec=grid_spec,
        out_shape=out_shape,
        scratch_shapes=scratch,
    )(emb_tm, ctx_tm, emb_tm, ctx_tm, h0f, c0f, h0b, c0b, Wf, bf, Wb, bb)


def kernel(inputs, context, decoder_hidden_state, decoder_cell_state, table,
           Wih_f, Whh_f, bih_f, bhh_f, Wih_b, Whh_b, bih_b, bhh_b):
    B, T = inputs.shape

    idx_tm = jnp.transpose(inputs).reshape(-1).astype(jnp.int32)
    emb_flat = _sc_gather(table, idx_tm)
    emb_tm = emb_flat.reshape(T, B, H)

    ctx_tm = jnp.transpose(context, (1, 0, 2))

    Wf = jnp.concatenate([Wih_f.T, Whh_f.T], axis=0)
    Wb = jnp.concatenate([Wih_b.T, Whh_b.T], axis=0)
    bf = (bih_f + bhh_f).reshape(1, -1)
    bb = (bih_b + bhh_b).reshape(1, -1)

    ys_f, ys_b, hf, cf, hb, cb = _lstm_tc(
        emb_tm, ctx_tm,
        decoder_hidden_state[0], decoder_cell_state[0],
        decoder_hidden_state[1], decoder_cell_state[1],
        Wf, bf, Wb, bb)

    out = jnp.transpose(jnp.concatenate([ys_f, ys_b], axis=-1), (1, 0, 2))
    h_n = jnp.stack([hf, hb], axis=0)
    c_n = jnp.stack([cf, cb], axis=0)
    return out, h_n, c_n


# speedup vs baseline: 1.0418x; 1.0418x over previous
import functools

import jax
import jax.numpy as jnp
from jax import lax
from jax.experimental import pallas as pl
from jax.experimental.pallas import tpu as pltpu
from jax.experimental.pallas import tpu_sc as plsc

H = 64
IDX_CHUNK = 128


def _sc_gather(table, idx):
    n = idx.shape[0]
    info = plsc.get_sparse_core_info()
    nw = info.num_cores * info.num_subcores
    assert n % nw == 0
    b_per_w = n // nw
    assert b_per_w % 8 == 0
    sizes = []
    left = b_per_w
    while left > 0:
        s = min(IDX_CHUNK, left)
        sizes.append(s)
        left -= s

    mesh = plsc.VectorSubcoreMesh(core_axis_name="c", subcore_axis_name="s")

    @functools.partial(
        pl.kernel,
        out_type=jax.ShapeDtypeStruct((n, H), jnp.float32),
        mesh=mesh,
        scratch_types=[
            pltpu.VMEM((b_per_w,), jnp.int32),
            pltpu.VMEM((b_per_w, H), jnp.float32),
            pltpu.SemaphoreType.DMA,
        ],
        compiler_params=pltpu.CompilerParams(use_tc_tiling_on_sc=False),
    )
    def k(table_hbm, idx_hbm, out_hbm, idx_v, rows_v, sem):
        wid = lax.axis_index("s") * info.num_cores + lax.axis_index("c")
        base = wid * b_per_w
        pltpu.sync_copy(idx_hbm.at[pl.ds(base, b_per_w)], idx_v)
        copies = []
        off = 0
        for s in sizes:
            copies.append(
                pltpu.async_copy(
                    table_hbm.at[idx_v.at[pl.ds(off, s)]],
                    rows_v.at[pl.ds(off, s)],
                    sem,
                )
            )
            off += s
        for c in copies:
            c.wait()
        pltpu.sync_copy(rows_v, out_hbm.at[pl.ds(base, b_per_w)])

    return k(table, idx)


def _lstm_tc(emb_tm, ctx_tm, h0f, c0f, h0b, c0b, Wf, bf, Wb, bb):
    T, B, _ = emb_tm.shape

    def body(emb_f, ctx_f, emb_b, ctx_b, h0f_r, c0f_r, h0b_r, c0b_r,
             wf_r, bf_r, wb_r, bb_r,
             out_f, out_b, hf_o, cf_o, hb_o, cb_o,
             hf_s, cf_s, hb_s, cb_s):
        t = pl.program_id(0)

        @pl.when(t == 0)
        def _():
            hf_s[:] = h0f_r[:]
            cf_s[:] = c0f_r[:]
            hb_s[:] = h0b_r[:]
            cb_s[:] = c0b_r[:]

        def step(emb, ctx, h, c, w, b):
            x = jnp.concatenate([emb, ctx, h], axis=-1)
            g = jnp.dot(x, w, preferred_element_type=jnp.float32) + b
            i = jax.nn.sigmoid(g[:, 0 * H:1 * H])
            f = jax.nn.sigmoid(g[:, 1 * H:2 * H])
            gg = jnp.tanh(g[:, 2 * H:3 * H])
            o = jax.nn.sigmoid(g[:, 3 * H:4 * H])
            c2 = f * c + i * gg
            h2 = o * jnp.tanh(c2)
            return h2, c2

        hf, cf = step(emb_f[0], ctx_f[0], hf_s[:], cf_s[:], wf_r[:], bf_r[:])
        hf_s[:] = hf
        cf_s[:] = cf
        out_f[0] = hf

        hb, cb = step(emb_b[0], ctx_b[0], hb_s[:], cb_s[:], wb_r[:], bb_r[:])
        hb_s[:] = hb
        cb_s[:] = cb
        out_b[0] = hb

        @pl.when(t == T - 1)
        def _():
            hf_o[:] = hf
            cf_o[:] = cf
            hb_o[:] = hb
            cb_o[:] = cb

    full = lambda shape: pl.BlockSpec(shape, lambda t: (0,) * len(shape))
    tspec = lambda w: pl.BlockSpec((1, B, w), lambda t: (t, 0, 0))
    rspec = lambda w: pl.BlockSpec((1, B, w), lambda t: (T - 1 - t, 0, 0))

    in_specs = [
            tspec(H), tspec(2 * H), rspec(H), rspec(2 * H),
            full((B, H)), full((B, H)), full((B, H)), full((B, H)),
            full((4 * H, 4 * H)), full((1, 4 * H)),
            full((4 * H, 4 * H)), full((1, 4 * H)),
    ]
    out_specs = [
        tspec(H), rspec(H),
        full((B, H)), full((B, H)), full((B, H)), full((B, H)),
    ]
    out_shape = [
        jax.ShapeDtypeStruct((T, B, H), jnp.float32),
        jax.ShapeDtypeStruct((T, B, H), jnp.float32),
        jax.ShapeDtypeStruct((B, H), jnp.float32),
        jax.ShapeDtypeStruct((B, H), jnp.float32),
        jax.ShapeDtypeStruct((B, H), jnp.float32),
        jax.ShapeDtypeStruct((B, H), jnp.float32),
    ]
    scratch = [pltpu.VMEM((B, H), jnp.float32)] * 4
    return pl.pallas_call(
        body,
        grid=(T,),
        in_specs=in_specs,
        out_specs=out_specs,
        out_shape=out_shape,
        scratch_shapes=scratch,
    )(emb_tm, ctx_tm, emb_tm, ctx_tm, h0f, c0f, h0b, c0b, Wf, bf, Wb, bb)


def kernel(inputs, context, decoder_hidden_state, decoder_cell_state, table,
           Wih_f, Whh_f, bih_f, bhh_f, Wih_b, Whh_b, bih_b, bhh_b):
    B, T = inputs.shape

    idx_tm = jnp.transpose(inputs).reshape(-1).astype(jnp.int32)
    emb_flat = _sc_gather(table, idx_tm)
    emb_tm = emb_flat.reshape(T, B, H)

    ctx_tm = jnp.transpose(context, (1, 0, 2))

    Wf = jnp.concatenate([Wih_f.T, Whh_f.T], axis=0)
    Wb = jnp.concatenate([Wih_b.T, Whh_b.T], axis=0)
    bf = (bih_f + bhh_f).reshape(1, -1)
    bb = (bih_b + bhh_b).reshape(1, -1)

    ys_f, ys_b, hf, cf, hb, cb = _lstm_tc(
        emb_tm, ctx_tm,
        decoder_hidden_state[0], decoder_cell_state[0],
        decoder_hidden_state[1], decoder_cell_state[1],
        Wf, bf, Wb, bb)

    out = jnp.transpose(jnp.concatenate([ys_f, ys_b], axis=-1), (1, 0, 2))
    h_n = jnp.stack([hf, hb], axis=0)
    c_n = jnp.stack([cf, cb], axis=0)
    return out, h_n, c_n
